# baseline (speedup 1.0000x reference)
import functools

import jax
import jax.numpy as jnp
import numpy as np
from jax.experimental import pallas as pl
from jax.experimental.pallas import tpu as pltpu

_B = 16
_T = 96
_PRED = 96
_TD = 144
_N = 7
_MARK = 4
_DM = 512
_NH = 8
_DH = _DM // _NH
_DFF = 1024
_NE = 4
_LAT = 64
_GATE = 6
_WIN = 25

_INTERP = False
_PREC = jax.lax.Precision.DEFAULT


def _b16(x):
    return x.astype(jnp.bfloat16).astype(jnp.float32)
_PREC_HI = jax.lax.Precision.HIGHEST


def _rowsum(x):
    L = x.shape[-1]
    if L == 4:
        a = x[:, 0:2] + x[:, 2:4]
        return a[:, 0:1] + a[:, 1:2]
    if L == 6:
        a = x[:, 0:2] + x[:, 4:6]
        b = a + x[:, 2:4]
        return b[:, 0:1] + b[:, 1:2]
    acc = x
    if L > 128 and L % 128 == 0:
        acc = x[:, 0:128]
        for i in range(1, L // 128):
            acc = acc + x[:, 128 * i:128 * (i + 1)]
    W = acc.shape[-1]
    assert W % 8 == 0
    a8 = acc[:, 0:8]
    for i in range(1, W // 8):
        a8 = a8 + acc[:, 8 * i:8 * (i + 1)]
    a4 = a8[:, 0:4] + a8[:, 4:8]
    a2 = a4[:, 0:2] + a4[:, 2:4]
    return a2[:, 0:1] + a2[:, 1:2]


def _mm_body(x_ref, w_ref, b_ref, o_ref, *, prec):
    o_ref[:] = jnp.dot(x_ref[:], w_ref[:],
                       preferred_element_type=jnp.float32,
                       precision=prec) + b_ref[:]


def _mm(x2, w, b, prec=None):
    M, K = x2.shape
    N = w.shape[1]
    body = functools.partial(_mm_body, prec=prec if prec is not None else _PREC)
    return pl.pallas_call(
        body,
        out_shape=jax.ShapeDtypeStruct((M, N), jnp.float32),
        interpret=_INTERP,
    )(x2, w, b.reshape(1, N))


def _ln_body(x_ref, r_ref, g_ref, b_ref, o_ref):
    h = x_ref[:] + r_ref[:]
    m = _rowsum(h) / float(_DM)
    d = h - m
    v = _rowsum(d * d) / float(_DM)
    o_ref[:] = d / jnp.sqrt(v + 1e-5) * g_ref[:] + b_ref[:]


def _ln_nores_body(x_ref, g_ref, b_ref, o_ref):
    h = x_ref[:]
    m = _rowsum(h) / float(_DM)
    d = h - m
    v = _rowsum(d * d) / float(_DM)
    o_ref[:] = d / jnp.sqrt(v + 1e-5) * g_ref[:] + b_ref[:]


def _ln(x2, res2, g, b):
    M, D = x2.shape
    if res2 is None:
        return pl.pallas_call(
            _ln_nores_body,
            out_shape=jax.ShapeDtypeStruct((M, D), jnp.float32),
            interpret=_INTERP,
        )(x2, g.reshape(1, D), b.reshape(1, D))
    return pl.pallas_call(
        _ln_body,
        out_shape=jax.ShapeDtypeStruct((M, D), jnp.float32),
        interpret=_INTERP,
    )(x2, res2, g.reshape(1, D), b.reshape(1, D))


def _attn_body(q_ref, k_ref, v_ref, o_ref, *, causal, scale):
    q = q_ref[0]
    k = k_ref[0]
    v = v_ref[0]
    s = jax.lax.dot_general(q, k, (((1,), (1,)), ((), ())),
                            preferred_element_type=jnp.float32, precision=_PREC) * scale
    if causal:
        lq, lk = s.shape
        row = jax.lax.broadcasted_iota(jnp.int32, (lq, lk), 0)
        col = jax.lax.broadcasted_iota(jnp.int32, (lq, lk), 1)
        s = jnp.where(col <= row + (lk - lq), s, -1e9)
    m = jnp.max(s, -1, keepdims=True)
    e = jnp.exp(s - m)
    a = e / _rowsum(e)
    o_ref[0] = jnp.dot(a, v, preferred_element_type=jnp.float32, precision=_PREC)


def _attn_core(q, k, v, causal):
    BH, Lq, dh = q.shape
    Lk = k.shape[1]
    body = functools.partial(_attn_body, causal=causal,
                             scale=1.0 / np.sqrt(dh))
    return pl.pallas_call(
        body,
        grid=(BH,),
        in_specs=[
            pl.BlockSpec((1, Lq, dh), lambda i: (i, 0, 0)),
            pl.BlockSpec((1, Lk, dh), lambda i: (i, 0, 0)),
            pl.BlockSpec((1, Lk, dh), lambda i: (i, 0, 0)),
        ],
        out_specs=pl.BlockSpec((1, Lq, dh), lambda i: (i, 0, 0)),
        out_shape=jax.ShapeDtypeStruct((BH, Lq, dh), jnp.float32),
        interpret=_INTERP,
    )(q, k, v)


def _attention(p, q3, k3, v3, causal):
    Bq, Lq, _ = q3.shape
    Lk = k3.shape[1]
    qkv_same = k3 is q3 and v3 is q3
    if qkv_same:
        wqkv = jnp.concatenate([p["q"]["w"], p["k"]["w"], p["v"]["w"]], 1)
        bqkv = jnp.concatenate([p["q"]["b"], p["k"]["b"], p["v"]["b"]], 0)
        qkv = _mm(q3.reshape(Bq * Lq, _DM), wqkv, bqkv)
        Q, K, V = qkv[:, :_DM], qkv[:, _DM:2 * _DM], qkv[:, 2 * _DM:]
    else:
        Q = _mm(q3.reshape(Bq * Lq, _DM), p["q"]["w"], p["q"]["b"])
        wkv = jnp.concatenate([p["k"]["w"], p["v"]["w"]], 1)
        bkv = jnp.concatenate([p["k"]["b"], p["v"]["b"]], 0)
        kv = _mm(k3.reshape(Bq * Lk, _DM), wkv, bkv)
        K, V = kv[:, :_DM], kv[:, _DM:]

    def heads(x2, L):
        return (x2.reshape(Bq, L, _NH, _DH).transpose(0, 2, 1, 3)
                .reshape(Bq * _NH, L, _DH))

    o = _attn_core(heads(Q, Lq), heads(K, Lk), heads(V, Lk), causal)
    o2 = (o.reshape(Bq, _NH, Lq, _DH).transpose(0, 2, 1, 3)
          .reshape(Bq * Lq, _DM))
    return _mm(o2, p["o"]["w"], p["o"]["b"]).reshape(Bq, Lq, _DM)


def _moe_body(x_ref, gw_ref, gb_ref, w1_ref, b1_ref, w2_ref, b2_ref, o_ref):
    xb = x_ref[:]
    TB = xb.shape[0]
    logits = jnp.dot(xb, gw_ref[:], preferred_element_type=jnp.float32, precision=_PREC) \
        + gb_ref[:]
    lm = jnp.max(logits, -1, keepdims=True)
    ex = jnp.exp(logits - lm)
    probs = ex / _rowsum(ex)
    idx = jax.lax.broadcasted_iota(jnp.int32, (TB, _NE), 1)
    v1 = jnp.max(probs, -1, keepdims=True)
    i1 = jnp.min(jnp.where(probs == v1, idx, _NE), -1, keepdims=True)
    pm = jnp.where(idx == i1, -1.0, probs)
    v2 = jnp.max(pm, -1, keepdims=True)
    i2 = jnp.min(jnp.where(pm == v2, idx, _NE), -1, keepdims=True)
    denom = v1 + v2
    wtok = (jnp.where(idx == i1, v1, 0.0)
            + jnp.where(idx == i2, v2, 0.0)) / denom
    wtok = _b16(wtok)
    acc = jnp.zeros((TB, _DM), jnp.float32)
    for e in range(_NE):
        h = jnp.dot(xb, w1_ref[e], preferred_element_type=jnp.float32, precision=_PREC) \
            + b1_ref[e]
        h = jax.nn.gelu(h)
        oe = jnp.dot(h, w2_ref[e], preferred_element_type=jnp.float32, precision=_PREC) \
            + b2_ref[e]
        acc = acc + wtok[:, e:e + 1] * _b16(oe)
    o_ref[:] = acc


def _moe(x2, p):
    M = x2.shape[0]
    TB = 384
    assert M % TB == 0
    grid = (M // TB,)
    return pl.pallas_call(
        _moe_body,
        grid=grid,
        in_specs=[
            pl.BlockSpec((TB, _DM), lambda i: (i, 0)),
            pl.BlockSpec((_DM, _NE), lambda i: (0, 0)),
            pl.BlockSpec((1, _NE), lambda i: (0, 0)),
            pl.BlockSpec((_NE, _DM, _DFF), lambda i: (0, 0, 0)),
            pl.BlockSpec((_NE, _DFF), lambda i: (0, 0)),
            pl.BlockSpec((_NE, _DFF, _DM), lambda i: (0, 0, 0)),
            pl.BlockSpec((_NE, _DM), lambda i: (0, 0)),
        ],
        out_specs=pl.BlockSpec((TB, _DM), lambda i: (i, 0)),
        out_shape=jax.ShapeDtypeStruct((M, _DM), jnp.float32),
        interpret=_INTERP,
    )(x2, p["gate"]["w"], p["gate"]["b"].reshape(1, _NE),
      p["w1"], p["b1"], p["w2"], p["b2"])


def _ptm_body(x_ref, gw_ref, gb_ref, ew_ref, eb_ref, o_ref):
    x = x_ref[:]
    logits = jnp.dot(x, gw_ref[:], preferred_element_type=jnp.float32, precision=_PREC) \
        + gb_ref[:]
    lm = jnp.max(logits, -1, keepdims=True)
    ex = jnp.exp(logits - lm)
    probs = ex / _rowsum(ex)
    probs = _b16(probs)
    e = _b16(jnp.dot(x, ew_ref[:], preferred_element_type=jnp.float32, precision=_PREC) + eb_ref[:])
    acc = jnp.zeros((x.shape[0], _T), jnp.float32)
    for g in range(_GATE):
        acc = acc + probs[:, g:g + 1] * e[:, g * _T:(g + 1) * _T]
    o_ref[:] = acc


def _ptm(xr, p):
    M = xr.shape[0]
    ewf = p["ew"].transpose(1, 0, 2).reshape(_T, _GATE * _T)
    ebt = jnp.tile(p["eb"], _GATE).reshape(1, _GATE * _T)
    return pl.pallas_call(
        _ptm_body,
        out_shape=jax.ShapeDtypeStruct((M, _T), jnp.float32),
        interpret=_INTERP,
    )(xr, p["gate"]["w"], p["gate"]["b"].reshape(1, _GATE), ewf, ebt)


def _vae_body(x_ref, wm_ref, bm_ref, wd_ref, bd_ref, o_ref):
    x = x_ref[:]
    mu = jnp.dot(x, wm_ref[:], preferred_element_type=jnp.float32, precision=_PREC) \
        + bm_ref[:]
    o_ref[:] = x + jnp.dot(mu, wd_ref[:],
                           preferred_element_type=jnp.float32, precision=_PREC) + bd_ref[:]


def _vae(x2, p):
    M = x2.shape[0]
    return pl.pallas_call(
        _vae_body,
        out_shape=jax.ShapeDtypeStruct((M, _DM), jnp.float32),
        interpret=_INTERP,
    )(x2, p["mu"]["w"], p["mu"]["b"].reshape(1, _LAT),
      p["dec"]["w"], p["dec"]["b"].reshape(1, _DM))


def _pe(length, d):
    pos = np.arange(length)[:, None].astype(np.float32)
    div = np.exp(np.arange(0, d, 2).astype(np.float32)
                 * -(np.log(10000.0) / d))
    pe = np.zeros((length, d), np.float32)
    pe[:, 0::2] = np.sin(pos * div)
    pe[:, 1::2] = np.cos(pos * div)
    return jnp.asarray(pe)[None]


_PE_ENC = _pe(_T, _DM)
_PE_DEC = _pe(_TD, _DM)


def _shifts(x):
    xp = jnp.concatenate([x[:, -1:, :], x, x[:, :1, :]], axis=1)
    B, L, C = x.shape
    return (xp[:, 0:L].reshape(B * L, C), xp[:, 1:L + 1].reshape(B * L, C),
            xp[:, 2:L + 2].reshape(B * L, C))


def _tok3_body(x0_ref, x1_ref, x2_ref, w0_ref, w1_ref, w2_ref, o_ref):
    t = jnp.dot(x0_ref[:], w0_ref[:], preferred_element_type=jnp.float32,
                precision=_PREC) \
        + jnp.dot(x1_ref[:], w1_ref[:], preferred_element_type=jnp.float32,
                  precision=_PREC)
    o_ref[:] = t + jnp.dot(x2_ref[:], w2_ref[:],
                           preferred_element_type=jnp.float32,
                           precision=_PREC)


def _token_embed(tokw, x):
    B, L, C = x.shape
    x0, x1, x2 = _shifts(x)
    out = pl.pallas_call(
        _tok3_body,
        out_shape=jax.ShapeDtypeStruct((B * L, _DM), jnp.float32),
        interpret=_INTERP,
    )(x0, x1, x2, tokw[0], tokw[1], tokw[2])
    return out.reshape(B, L, _DM)


def _emb4_body(x0_ref, x1_ref, x2_ref, xm_ref, w0_ref, w1_ref, w2_ref,
               wm_ref, bm_ref, o_ref):
    t = jnp.dot(x0_ref[:], w0_ref[:], preferred_element_type=jnp.float32,
                precision=_PREC) \
        + jnp.dot(x1_ref[:], w1_ref[:], preferred_element_type=jnp.float32,
                  precision=_PREC)
    t = t + jnp.dot(x2_ref[:], w2_ref[:], preferred_element_type=jnp.float32,
                    precision=_PREC)
    o_ref[:] = t + (jnp.dot(xm_ref[:], wm_ref[:],
                            preferred_element_type=jnp.float32,
                            precision=_PREC) + bm_ref[:])


def _data_emb(p, pe, x, x_mark):
    B, L, C = x.shape
    x0, x1, x2 = _shifts(x)
    tokw = p["tok"]["w"]
    out = pl.pallas_call(
        _emb4_body,
        out_shape=jax.ShapeDtypeStruct((B * L, _DM), jnp.float32),
        interpret=_INTERP,
    )(x0, x1, x2, x_mark.reshape(B * L, _MARK), tokw[0], tokw[1], tokw[2],
      p["temp"]["w"], p["temp"]["b"].reshape(1, _DM))
    return out.reshape(B, L, _DM) + pe[:, :L]


def _mixed_embedding(x, x_mark, p_inv, p_emb, pe, p_proj):
    B, L, C = x.shape
    xi = jnp.concatenate([x.transpose(0, 2, 1),
                          x_mark.transpose(0, 2, 1)], 1)
    Mi = xi.shape[1]
    inv = _mm(xi.reshape(B * Mi, L), p_inv["proj"]["w"], p_inv["proj"]["b"])
    rec = _mm(inv, p_proj["w"], p_proj["b"]).reshape(B, Mi, L)
    rec = rec[:, :C, :].transpose(0, 2, 1)
    return _data_emb(p_emb, pe, rec + x, x_mark)


def _ma_body(z_ref, o_ref, c_ref):
    c_ref[0:1] = z_ref[0:1]

    def step(i, carry):
        c_ref[pl.ds(i, 1)] = c_ref[pl.ds(i - 1, 1)] + z_ref[pl.ds(i, 1)]
        return carry

    jax.lax.fori_loop(1, 121, step, 0)
    o_ref[:] = (c_ref[_WIN:121] - c_ref[0:_T]) / float(_WIN)


def _moving_avg_rows(x):
    pad = _WIN // 2
    xp = jnp.pad(x, ((0, 0), (pad, pad), (0, 0)), mode='reflect')
    Lp = _T + 2 * pad
    XP = xp.transpose(1, 0, 2).reshape(Lp, _B * _N)
    z = jnp.concatenate([jnp.zeros((1, _B * _N), jnp.float32), XP], 0)
    out = pl.pallas_call(
        _ma_body,
        out_shape=jax.ShapeDtypeStruct((_T, _B * _N), jnp.float32),
        scratch_shapes=[pltpu.VMEM((121, _B * _N), jnp.float32)],
        interpret=_INTERP,
    )(z)
    return out.T


def _enc_layer(p, x3):
    B, L, _ = x3.shape
    a = _attention(p["attn"], x3, x3, x3, False)
    x2 = _ln(a.reshape(B * L, _DM), x3.reshape(B * L, _DM),
             p["n1"]["g"], p["n1"]["b"])
    m = _moe(x2, p["moe"])
    out = _ln(m, x2, p["n2"]["g"], p["n2"]["b"])
    return out.reshape(B, L, _DM)


def _dec_layer(p, x3, mem3):
    B, L, _ = x3.shape
    a = _attention(p["sa"], x3, x3, x3, True)
    x2 = _ln(a.reshape(B * L, _DM), x3.reshape(B * L, _DM),
             p["n1"]["g"], p["n1"]["b"])
    x3 = x2.reshape(B, L, _DM)
    c = _attention(p["ca"], x3, mem3, mem3, False)
    x2 = _ln(c.reshape(B * L, _DM), x2, p["n2"]["g"], p["n2"]["b"])
    m = _moe(x2, p["moe"])
    out = _ln(m, x2, p["n3"]["g"], p["n3"]["b"])
    return out.reshape(B, L, _DM)


def kernel(x_enc, x_mark_enc, x_dec, x_mark_dec, params):
    p = params
    tr_rows = _moving_avg_rows(x_enc)
    y = _ptm(tr_rows, p["ptm"])
    trend = y.reshape(_B, _N, _T).transpose(0, 2, 1)
    trend_emb = _token_embed(p["tok"]["w"], trend)

    enc = _mixed_embedding(x_enc, x_mark_enc, p["enc_inv"], p["enc_emb"],
                           _PE_ENC, p["enc_proj"]) + trend_emb
    for lp in p["enc_layers"]:
        enc = _enc_layer(lp, enc)
    enc2 = _ln(enc.reshape(_B * _T, _DM), None,
               p["enc_norm"]["g"], p["enc_norm"]["b"])
    enc2 = _vae(enc2, p["vae"])
    enc = enc2.reshape(_B, _T, _DM)

    dec = _mixed_embedding(x_dec, x_mark_dec, p["dec_inv"], p["dec_emb"],
                           _PE_DEC, p["dec_proj_emb"])
    for lp in p["dec_layers"]:
        dec = _dec_layer(lp, dec, enc)
    dec2 = _ln(dec.reshape(_B * _TD, _DM), None,
               p["dec_norm"]["g"], p["dec_norm"]["b"])
    out = _mm(dec2, p["out_proj"]["w"], p["out_proj"]["b"])
    out = out.reshape(_B, _TD, -1)
    return out[:, -_PRED:, :]
